# baseline (speedup 1.0000x reference)
import jax
import jax.numpy as jnp
from jax import lax
from jax.experimental import pallas as pl
from jax.experimental.pallas import tpu as pltpu
from jax.experimental.pallas import tpu_sc as plsc

N_NODES = 10000
N_EDGES = 160000
D_IN = 256
D_OUT = 256
HALF = D_OUT // 2

NUM_SC = 2
NUM_TILES = 16
CHUNK = 128
N_CHUNKS = N_EDGES // CHUNK
CHUNKS_PER_TILE = (N_CHUNKS + NUM_TILES - 1) // NUM_TILES
ACC_ROWS = N_NODES + 8
ROWS_MAIN = 624
ROWS_LAST = N_NODES - (NUM_TILES - 1) * ROWS_MAIN


def _matmul_half(x, W):
    RB = 1000

    def body(x_ref, w_ref, o_ref):
        o_ref[0] = jnp.dot(x_ref[...], w_ref[0], preferred_element_type=jnp.float32)

    w2 = W.reshape(D_IN, 2, HALF).transpose(1, 0, 2)
    return pl.pallas_call(
        body,
        grid=(NUM_SC, N_NODES // RB),
        in_specs=[
            pl.BlockSpec((RB, D_IN), lambda c, i: (i, 0)),
            pl.BlockSpec((1, D_IN, HALF), lambda c, i: (c, 0, 0)),
        ],
        out_specs=pl.BlockSpec((1, RB, HALF), lambda c, i: (c, i, 0)),
        out_shape=jax.ShapeDtypeStruct((NUM_SC, N_NODES, HALF), jnp.float32),
    )(x, w2)


CH_MAIN = 80
CH_LAST_LOAD = 56
CH_LAST = N_CHUNKS - (NUM_TILES - 1) * CH_MAIN
N_CHUNKS_PAD = (NUM_TILES - 1) * CH_MAIN + CH_LAST_LOAD


def _sc_aggregate(h2, src2, dst2, b):
    mesh = plsc.VectorSubcoreMesh(core_axis_name="c", subcore_axis_name="s")

    def body(h_hbm, src_hbm, dstf_hbm, b_hbm, out_hbm,
             acc, srcb, bbuf, rows0, rows1, dst0, dst1,
             sem, semg0, semg1, semd0, semd1, sems0, sems1):
        c = lax.axis_index("c")
        s = lax.axis_index("s")
        h_c = h_hbm.at[c]
        c0 = s * CH_MAIN

        cp_s_big = pltpu.make_async_copy(src_hbm.at[pl.ds(c0, CH_MAIN)], srcb, sem)
        cp_s_sml = pltpu.make_async_copy(
            src_hbm.at[pl.ds(c0, CH_LAST_LOAD)], srcb.at[pl.ds(0, CH_LAST_LOAD)], sem
        )

        @pl.when(s < NUM_TILES - 1)
        def _():
            cp_s_big.start()

        @pl.when(s == NUM_TILES - 1)
        def _():
            cp_s_sml.start()

        @pl.when(c == 0)
        def _():
            pltpu.sync_copy(b_hbm.at[pl.ds(0, HALF)], bbuf)

        @pl.when(c == 1)
        def _():
            pltpu.sync_copy(b_hbm.at[pl.ds(HALF, HALF)], bbuf)

        @pl.loop(0, CHUNK)
        def _(r):
            @pl.loop(0, HALF, step=16)
            def _(cc):
                rows0[r, pl.ds(cc, 16)] = bbuf[pl.ds(cc, 16)]

        row0 = s * ROWS_MAIN

        def init_rows(nrows):
            full = nrows // CHUNK
            rem = nrows - full * CHUNK

            @pl.loop(0, full)
            def _(k):
                pltpu.sync_copy(rows0, acc.at[pl.ds(row0 + k * CHUNK, CHUNK)])

            if rem:
                pltpu.sync_copy(
                    rows0.at[pl.ds(0, rem)], acc.at[pl.ds(row0 + full * CHUNK, rem)]
                )

        @pl.when(s < NUM_TILES - 1)
        def _():
            init_rows(ROWS_MAIN)

        @pl.when(s == NUM_TILES - 1)
        def _():
            init_rows(ROWS_LAST)

        @pl.when(s < NUM_TILES - 1)
        def _():
            cp_s_big.wait()

        @pl.when(s == NUM_TILES - 1)
        def _():
            cp_s_sml.wait()

        plsc.subcore_barrier()

        def start_g(j, rb, sg):
            pltpu.make_async_copy(h_c.at[srcb.at[j]], rb, sg).start()

        def wait_g(rb, sg):
            pltpu.make_async_copy(h_c.at[srcb.at[0]], rb, sg).wait()

        def start_d(g, db, sd):
            e0 = (c0 + g) * CHUNK
            pltpu.make_async_copy(dstf_hbm.at[pl.ds(e0, CHUNK)], db, sd).start()

        def wait_d(db, sd):
            pltpu.make_async_copy(dstf_hbm.at[pl.ds(0, CHUNK)], db, sd).wait()

        def run_chunks(n):
            start_d(0, dst0, semd0)
            start_d(1, dst1, semd1)
            start_g(0, rows0, semg0)
            start_g(1, rows1, semg1)

            @pl.loop(0, n - 2, step=2)
            def _(g):
                wait_g(rows0, semg0)
                wait_d(dst0, semd0)
                sc0 = pltpu.async_copy(rows0, acc.at[dst0], sems0, add=True)
                wait_g(rows1, semg1)
                wait_d(dst1, semd1)
                sc1 = pltpu.async_copy(rows1, acc.at[dst1], sems1, add=True)
                sc0.wait()
                start_g(g + 2, rows0, semg0)
                start_d(g + 2, dst0, semd0)
                sc1.wait()
                start_g(g + 3, rows1, semg1)
                start_d(g + 3, dst1, semd1)

            wait_g(rows0, semg0)
            wait_d(dst0, semd0)
            sc0 = pltpu.async_copy(rows0, acc.at[dst0], sems0, add=True)
            wait_g(rows1, semg1)
            wait_d(dst1, semd1)
            sc1 = pltpu.async_copy(rows1, acc.at[dst1], sems1, add=True)
            sc0.wait()
            sc1.wait()

        @pl.when(s < NUM_TILES - 1)
        def _():
            run_chunks(CH_MAIN)

        @pl.when(s == NUM_TILES - 1)
        def _():
            run_chunks(CH_LAST)

        plsc.subcore_barrier()

        def writeout(nrows):
            @pl.when(c == 0)
            def _():
                pltpu.sync_copy(
                    acc.at[pl.ds(row0, nrows)],
                    out_hbm.at[pl.ds(row0, nrows), pl.ds(0, HALF)],
                )

            @pl.when(c == 1)
            def _():
                pltpu.sync_copy(
                    acc.at[pl.ds(row0, nrows)],
                    out_hbm.at[pl.ds(row0, nrows), pl.ds(HALF, HALF)],
                )

        @pl.when(s < NUM_TILES - 1)
        def _():
            writeout(ROWS_MAIN)

        @pl.when(s == NUM_TILES - 1)
        def _():
            writeout(ROWS_LAST)

    kern = pl.kernel(
        body,
        out_type=jax.ShapeDtypeStruct((N_NODES, D_OUT), jnp.float32),
        mesh=mesh,
        scratch_types=[
            pltpu.VMEM_SHARED((ACC_ROWS, HALF), jnp.float32),
            pltpu.VMEM((CH_MAIN, CHUNK), jnp.int32),
            pltpu.VMEM((HALF,), jnp.float32),
            pltpu.VMEM((CHUNK, HALF), jnp.float32),
            pltpu.VMEM((CHUNK, HALF), jnp.float32),
            pltpu.VMEM((CHUNK,), jnp.int32),
            pltpu.VMEM((CHUNK,), jnp.int32),
            pltpu.SemaphoreType.DMA,
            pltpu.SemaphoreType.DMA,
            pltpu.SemaphoreType.DMA,
            pltpu.SemaphoreType.DMA,
            pltpu.SemaphoreType.DMA,
            pltpu.SemaphoreType.DMA,
            pltpu.SemaphoreType.DMA,
        ],
    )
    return kern(h2, src2, dst2, b)


def kernel(x, edge, W, b):
    pad = jnp.zeros(((N_CHUNKS_PAD - N_CHUNKS) * CHUNK,), jnp.int32)
    src2 = jnp.concatenate(
        [edge[0].astype(jnp.int32), pad]).reshape(N_CHUNKS_PAD, CHUNK)
    dstf = jnp.concatenate([edge[1].astype(jnp.int32), pad])
    h2 = _matmul_half(x, W)
    return _sc_aggregate(h2, src2, dstf, b)


# speedup vs baseline: 1.2623x; 1.2623x over previous
import jax
import jax.numpy as jnp
from jax import lax
from jax.experimental import pallas as pl
from jax.experimental.pallas import tpu as pltpu
from jax.experimental.pallas import tpu_sc as plsc

N_NODES = 10000
N_EDGES = 160000
D_IN = 256
D_OUT = 256
HALF = D_OUT // 2

NUM_SC = 2
NUM_TILES = 16
CHUNK = 128
N_CHUNKS = N_EDGES // CHUNK
CHUNKS_PER_TILE = (N_CHUNKS + NUM_TILES - 1) // NUM_TILES
ACC_ROWS = N_NODES + 8
ROWS_MAIN = 624
ROWS_LAST = N_NODES - (NUM_TILES - 1) * ROWS_MAIN


def _matmul_half(x, W):
    RB = 1000

    def body(x_ref, w_ref, o_ref):
        o_ref[0] = jnp.dot(x_ref[...], w_ref[0], preferred_element_type=jnp.float32)

    w2 = W.reshape(D_IN, 2, HALF).transpose(1, 0, 2)
    return pl.pallas_call(
        body,
        grid=(NUM_SC, N_NODES // RB),
        in_specs=[
            pl.BlockSpec((RB, D_IN), lambda c, i: (i, 0)),
            pl.BlockSpec((1, D_IN, HALF), lambda c, i: (c, 0, 0)),
        ],
        out_specs=pl.BlockSpec((1, RB, HALF), lambda c, i: (c, i, 0)),
        out_shape=jax.ShapeDtypeStruct((NUM_SC, N_NODES, HALF), jnp.float32),
    )(x, w2)


CH_MAIN = 80
CH_LAST = N_CHUNKS - (NUM_TILES - 1) * CH_MAIN


def _sc_aggregate(h2, src, dst, b):
    mesh = plsc.VectorSubcoreMesh(core_axis_name="c", subcore_axis_name="s")

    def body(h_hbm, srcf_hbm, dstf_hbm, b_hbm, out_hbm,
             acc, srcb, bbuf, rows0, rows1, dst0, dst1,
             sem, semg0, semg1, semd0, semd1):
        c = lax.axis_index("c")
        s = lax.axis_index("s")
        h_c = h_hbm.at[c]
        c0 = s * CH_MAIN

        e00 = c0 * CHUNK
        cp_s_big = pltpu.make_async_copy(
            srcf_hbm.at[pl.ds(e00, CH_MAIN * CHUNK)], srcb, sem)
        cp_s_sml = pltpu.make_async_copy(
            srcf_hbm.at[pl.ds(e00, CH_LAST * CHUNK)],
            srcb.at[pl.ds(0, CH_LAST * CHUNK)], sem)

        @pl.when(s < NUM_TILES - 1)
        def _():
            cp_s_big.start()

        @pl.when(s == NUM_TILES - 1)
        def _():
            cp_s_sml.start()

        @pl.when(c == 0)
        def _():
            pltpu.sync_copy(b_hbm.at[pl.ds(0, HALF)], bbuf)

        @pl.when(c == 1)
        def _():
            pltpu.sync_copy(b_hbm.at[pl.ds(HALF, HALF)], bbuf)

        @pl.loop(0, CHUNK)
        def _(r):
            @pl.loop(0, HALF, step=16)
            def _(cc):
                rows0[r, pl.ds(cc, 16)] = bbuf[pl.ds(cc, 16)]

        row0 = s * ROWS_MAIN

        def init_rows(nrows):
            full = nrows // CHUNK
            rem = nrows - full * CHUNK

            @pl.loop(0, full)
            def _(k):
                pltpu.sync_copy(rows0, acc.at[pl.ds(row0 + k * CHUNK, CHUNK)])

            if rem:
                pltpu.sync_copy(
                    rows0.at[pl.ds(0, rem)], acc.at[pl.ds(row0 + full * CHUNK, rem)]
                )

        @pl.when(s < NUM_TILES - 1)
        def _():
            init_rows(ROWS_MAIN)

        @pl.when(s == NUM_TILES - 1)
        def _():
            init_rows(ROWS_LAST)

        @pl.when(s < NUM_TILES - 1)
        def _():
            cp_s_big.wait()

        @pl.when(s == NUM_TILES - 1)
        def _():
            cp_s_sml.wait()

        plsc.subcore_barrier()

        def start_g(j, rb, sg):
            pltpu.make_async_copy(
                h_c.at[srcb.at[pl.ds(j * CHUNK, CHUNK)]], rb, sg).start()

        def wait_g(rb, sg):
            pltpu.make_async_copy(
                h_c.at[srcb.at[pl.ds(0, CHUNK)]], rb, sg).wait()

        def start_d(g, db, sd):
            e0 = (c0 + g) * CHUNK
            pltpu.make_async_copy(dstf_hbm.at[pl.ds(e0, CHUNK)], db, sd).start()

        def wait_d(db, sd):
            pltpu.make_async_copy(dstf_hbm.at[pl.ds(0, CHUNK)], db, sd).wait()

        def run_chunks(n):
            start_d(0, dst0, semd0)
            start_d(1, dst1, semd1)
            start_g(0, rows0, semg0)
            start_g(1, rows1, semg1)

            @pl.loop(0, n - 2, step=2)
            def _(g):
                wait_g(rows0, semg0)
                wait_d(dst0, semd0)
                pltpu.sync_copy(rows0, acc.at[dst0], add=True)
                start_g(g + 2, rows0, semg0)
                start_d(g + 2, dst0, semd0)

                wait_g(rows1, semg1)
                wait_d(dst1, semd1)
                pltpu.sync_copy(rows1, acc.at[dst1], add=True)
                start_g(g + 3, rows1, semg1)
                start_d(g + 3, dst1, semd1)

            wait_g(rows0, semg0)
            wait_d(dst0, semd0)
            pltpu.sync_copy(rows0, acc.at[dst0], add=True)
            wait_g(rows1, semg1)
            wait_d(dst1, semd1)
            pltpu.sync_copy(rows1, acc.at[dst1], add=True)

        @pl.when(s < NUM_TILES - 1)
        def _():
            run_chunks(CH_MAIN)

        @pl.when(s == NUM_TILES - 1)
        def _():
            run_chunks(CH_LAST)

        plsc.subcore_barrier()

        def writeout(nrows):
            @pl.when(c == 0)
            def _():
                pltpu.sync_copy(
                    acc.at[pl.ds(row0, nrows)],
                    out_hbm.at[pl.ds(row0, nrows), pl.ds(0, HALF)],
                )

            @pl.when(c == 1)
            def _():
                pltpu.sync_copy(
                    acc.at[pl.ds(row0, nrows)],
                    out_hbm.at[pl.ds(row0, nrows), pl.ds(HALF, HALF)],
                )

        @pl.when(s < NUM_TILES - 1)
        def _():
            writeout(ROWS_MAIN)

        @pl.when(s == NUM_TILES - 1)
        def _():
            writeout(ROWS_LAST)

    kern = pl.kernel(
        body,
        out_type=jax.ShapeDtypeStruct((N_NODES, D_OUT), jnp.float32),
        mesh=mesh,
        scratch_types=[
            pltpu.VMEM_SHARED((ACC_ROWS, HALF), jnp.float32),
            pltpu.VMEM((CH_MAIN * CHUNK,), jnp.int32),
            pltpu.VMEM((HALF,), jnp.float32),
            pltpu.VMEM((CHUNK, HALF), jnp.float32),
            pltpu.VMEM((CHUNK, HALF), jnp.float32),
            pltpu.VMEM((CHUNK,), jnp.int32),
            pltpu.VMEM((CHUNK,), jnp.int32),
            pltpu.SemaphoreType.DMA,
            pltpu.SemaphoreType.DMA,
            pltpu.SemaphoreType.DMA,
            pltpu.SemaphoreType.DMA,
            pltpu.SemaphoreType.DMA,
        ],
    )
    return kern(h2, src, dst, b)


def kernel(x, edge, W, b):
    src = edge[0].astype(jnp.int32)
    dst = edge[1].astype(jnp.int32)
    h2 = _matmul_half(x, W)
    return _sc_aggregate(h2, src, dst, b)
